# baseline (speedup 1.0000x reference)
import jax
import jax.numpy as jnp
from jax import lax
from jax.experimental import pallas as pl
from jax.experimental.pallas import tpu as pltpu

B, H = 128, 1024
G = 8
HB = 4 * H // G
RPS = B // G
ND = 8

_dn = (((1,), (1,)), ((), ()))


def _w_copy(w_hbm, w_buf, sems, blk):
    return pltpu.make_async_copy(
        w_hbm.at[pl.ds(blk * HB, HB)],
        w_buf.at[lax.rem(blk, ND)],
        sems.at[lax.rem(blk, ND)],
    )


def _body(x_ref, h0_ref, c0_ref, bih_ref, bhh_ref,
          table_ref, whh_hbm, wih_hbm, h_out, c_out,
          acc_ref, emb_ref, whh_buf, wih_buf, sem, semh, semi):
    k = pl.program_id(0)

    @pl.when(k == 0)
    def _prime_whh():
        for blk in range(ND):
            _w_copy(whh_hbm, whh_buf, semh, blk).start()

    @pl.when((k >= 2) & (k <= G + 1))
    def _issue_wih():
        _w_copy(wih_hbm, wih_buf, semi, k - 2).start()

    @pl.when(k < G)
    def _phase_a():
        for r in range(RPS):
            b = k * RPS + r
            pltpu.make_async_copy(
                table_ref.at[pl.ds(x_ref[b], 1)],
                emb_ref.at[pl.ds(b, 1)],
                sem,
            ).start()
        _w_copy(whh_hbm, whh_buf, semh, k).wait()
        whh = whh_buf[lax.rem(k, ND)]
        acc = lax.dot_general(h0_ref[...], whh, _dn,
                              preferred_element_type=jnp.float32)
        acc_ref[:, pl.ds(k * HB, HB)] = acc + bih_ref[...] + bhh_ref[...]

    @pl.when(k >= G)
    def _phase_b():
        j = k - G

        @pl.when(j == 0)
        def _wait_gather():
            for r in range(B):
                pltpu.make_async_copy(
                    table_ref.at[pl.ds(x_ref[r], 1)],
                    emb_ref.at[pl.ds(r, 1)],
                    sem,
                ).wait()

        _w_copy(wih_hbm, wih_buf, semi, j).wait()
        wih = wih_buf[lax.rem(j, ND)]
        pre = lax.dot_general(emb_ref[...], wih, _dn,
                              preferred_element_type=jnp.float32)
        pre += acc_ref[:, pl.ds(j * HB, HB)]
        quarter = j // (G // 4)
        act = jnp.where(quarter == 2, jnp.tanh(pre), jax.nn.sigmoid(pre))
        acc_ref[:, pl.ds(j * HB, HB)] = act

        @pl.when(quarter == 3)
        def _finish():
            col = (j - 3 * (G // 4)) * HB
            i = acc_ref[:, pl.ds(col, HB)]
            f = acc_ref[:, pl.ds(H + col, HB)]
            g = acc_ref[:, pl.ds(2 * H + col, HB)]
            c = f * c0_ref[:, pl.ds(col, HB)] + i * g
            c_out[:, pl.ds(col, HB)] = c
            h_out[:, pl.ds(col, HB)] = act * jnp.tanh(c)


def _mk(interpret=False):
    return pl.pallas_call(
        _body,
        grid=(2 * G,),
        in_specs=[
            pl.BlockSpec(memory_space=pltpu.SMEM),
            pl.BlockSpec((B, H), lambda k: (0, 0)),
            pl.BlockSpec((B, H), lambda k: (0, 0)),
            pl.BlockSpec((1, HB),
                         lambda k: (0, jnp.minimum(k, G - 1))),
            pl.BlockSpec((1, HB),
                         lambda k: (0, jnp.minimum(k, G - 1))),
            pl.BlockSpec(memory_space=pl.ANY),
            pl.BlockSpec(memory_space=pl.ANY),
            pl.BlockSpec(memory_space=pl.ANY),
        ],
        out_specs=[
            pl.BlockSpec((B, H), lambda k: (0, 0)),
            pl.BlockSpec((B, H), lambda k: (0, 0)),
        ],
        out_shape=[
            jax.ShapeDtypeStruct((B, H), jnp.float32),
            jax.ShapeDtypeStruct((B, H), jnp.float32),
        ],
        scratch_shapes=[
            pltpu.VMEM((B, 4 * H), jnp.float32),
            pltpu.VMEM((B, H), jnp.float32),
            pltpu.VMEM((ND, HB, H), jnp.float32),
            pltpu.VMEM((ND, HB, H), jnp.float32),
            pltpu.SemaphoreType.DMA,
            pltpu.SemaphoreType.DMA((ND,)),
            pltpu.SemaphoreType.DMA((ND,)),
        ],
        compiler_params=pltpu.CompilerParams(
            dimension_semantics=("arbitrary",)),
        interpret=interpret,
    )


_lstm = _mk()


def kernel(x, hidden, cell, table, W_ih, W_hh, b_ih, b_hh):
    h, c = _lstm(x, hidden[0], cell[0],
                 b_ih.reshape(1, 4 * H), b_hh.reshape(1, 4 * H),
                 table, W_hh, W_ih)
    return (h[None], h[None], c[None])


# speedup vs baseline: 1.1409x; 1.1409x over previous
import jax
import jax.numpy as jnp
from jax import lax
from jax.experimental import pallas as pl
from jax.experimental.pallas import tpu as pltpu

B, H = 128, 1024
G = 8
HB = 4 * H // G
RPS = B // G
ND = 4

_dn = (((1,), (1,)), ((), ()))


def _w_copy(w_hbm, w_buf, sems, blk):
    return pltpu.make_async_copy(
        w_hbm.at[pl.ds(blk * HB, HB)],
        w_buf.at[lax.rem(blk, ND)],
        sems.at[lax.rem(blk, ND)],
    )


def _body(x_ref, h0_ref, c0_ref, bih_ref, bhh_ref,
          table_ref, whh_hbm, wih_hbm, h_out, c_out,
          acc_ref, emb_ref, whh_buf, wih_buf, sem, semh, semi):
    k = pl.program_id(0)

    @pl.when(k == 0)
    def _prime_whh():
        for blk in range(ND):
            _w_copy(whh_hbm, whh_buf, semh, blk).start()

    @pl.when((k >= 1) & (k <= G - ND))
    def _issue_whh():
        _w_copy(whh_hbm, whh_buf, semh, k + ND - 1).start()

    @pl.when((k >= G - 3) & (k <= 2 * G - 4))
    def _issue_wih():
        _w_copy(wih_hbm, wih_buf, semi, k - (G - 3)).start()

    @pl.when(k < G)
    def _phase_a():
        for r in range(RPS):
            b = k * RPS + r
            pltpu.make_async_copy(
                table_ref.at[pl.ds(x_ref[b], 1)],
                emb_ref.at[pl.ds(b, 1)],
                sem,
            ).start()
        _w_copy(whh_hbm, whh_buf, semh, k).wait()
        whh = whh_buf[lax.rem(k, ND)]
        acc = lax.dot_general(h0_ref[...], whh, _dn,
                              preferred_element_type=jnp.float32)
        acc_ref[:, pl.ds(k * HB, HB)] = acc + bih_ref[...] + bhh_ref[...]

    @pl.when(k >= G)
    def _phase_b():
        j = k - G

        @pl.when(j == 0)
        def _wait_gather():
            for r in range(B):
                pltpu.make_async_copy(
                    table_ref.at[pl.ds(x_ref[r], 1)],
                    emb_ref.at[pl.ds(r, 1)],
                    sem,
                ).wait()

        _w_copy(wih_hbm, wih_buf, semi, j).wait()
        wih = wih_buf[lax.rem(j, ND)]
        pre = lax.dot_general(emb_ref[...], wih, _dn,
                              preferred_element_type=jnp.float32)
        pre += acc_ref[:, pl.ds(j * HB, HB)]
        quarter = j // (G // 4)
        act = jnp.where(quarter == 2, jnp.tanh(pre), jax.nn.sigmoid(pre))
        acc_ref[:, pl.ds(j * HB, HB)] = act

        @pl.when(quarter == 3)
        def _finish():
            col = (j - 3 * (G // 4)) * HB
            i = acc_ref[:, pl.ds(col, HB)]
            f = acc_ref[:, pl.ds(H + col, HB)]
            g = acc_ref[:, pl.ds(2 * H + col, HB)]
            c = f * c0_ref[:, pl.ds(col, HB)] + i * g
            c_out[:, pl.ds(col, HB)] = c
            h_out[:, pl.ds(col, HB)] = act * jnp.tanh(c)


def _mk(interpret=False):
    return pl.pallas_call(
        _body,
        grid=(2 * G,),
        in_specs=[
            pl.BlockSpec(memory_space=pltpu.SMEM),
            pl.BlockSpec((B, H), lambda k: (0, 0)),
            pl.BlockSpec((B, H), lambda k: (0, 0)),
            pl.BlockSpec((1, HB),
                         lambda k: (0, jnp.minimum(k, G - 1))),
            pl.BlockSpec((1, HB),
                         lambda k: (0, jnp.minimum(k, G - 1))),
            pl.BlockSpec(memory_space=pl.ANY),
            pl.BlockSpec(memory_space=pl.ANY),
            pl.BlockSpec(memory_space=pl.ANY),
        ],
        out_specs=[
            pl.BlockSpec((B, H), lambda k: (0, 0)),
            pl.BlockSpec((B, H), lambda k: (0, 0)),
        ],
        out_shape=[
            jax.ShapeDtypeStruct((B, H), jnp.float32),
            jax.ShapeDtypeStruct((B, H), jnp.float32),
        ],
        scratch_shapes=[
            pltpu.VMEM((B, 4 * H), jnp.float32),
            pltpu.VMEM((B, H), jnp.float32),
            pltpu.VMEM((ND, HB, H), jnp.float32),
            pltpu.VMEM((ND, HB, H), jnp.float32),
            pltpu.SemaphoreType.DMA,
            pltpu.SemaphoreType.DMA((ND,)),
            pltpu.SemaphoreType.DMA((ND,)),
        ],
        compiler_params=pltpu.CompilerParams(
            dimension_semantics=("arbitrary",)),
        interpret=interpret,
    )


_lstm = _mk()


def kernel(x, hidden, cell, table, W_ih, W_hh, b_ih, b_hh):
    h, c = _lstm(x, hidden[0], cell[0],
                 b_ih.reshape(1, 4 * H), b_hh.reshape(1, 4 * H),
                 table, W_hh, W_ih)
    return (h[None], h[None], c[None])
